# baseline (device time: 71937 ns/iter reference)
import jax
import jax.numpy as jnp
from jax import lax
from jax.experimental import pallas as pl
from jax.experimental.pallas import tpu as pltpu

N_DEV = 4
X_DT = jnp.float8_e4m3fn
W_DT = jnp.float8_e5m2


def _cast(a, dt):
    m, k = a.shape

    def body(a_ref, o_ref):
        o_ref[...] = a_ref[...].astype(dt)

    return pl.pallas_call(
        body,
        grid=(4,),
        in_specs=[pl.BlockSpec((m // 4, k), lambda i: (i, 0))],
        out_specs=pl.BlockSpec((m // 4, k), lambda i: (i, 0)),
        out_shape=jax.ShapeDtypeStruct((m, k), dt),
    )(a)


def _cast_x(a):
    return _cast(a, X_DT)


def kernel(x, w_mat, scale_x, scale_w):
    m_per, k = x.shape
    _, n = w_mat.shape
    n_per = n // N_DEV

    xq = _cast(x, X_DT)
    wq_mat = _cast(w_mat, W_DT)
    my_arr = jnp.full((1,), lax.axis_index("i"), jnp.int32)

    def body(my_ref, x_ref, w_ref, sx_ref, sw_ref, out_ref,
             sendbuf, recvbuf, send_sems, recv_sems):
        j = pl.program_id(0)
        del my_ref
        my = lax.axis_index("i")
        tgt = lax.rem(my + 1 + j, N_DEV)
        s = sx_ref[0] * sw_ref[0]

        blk = jnp.maximum(
            jnp.dot(x_ref[...], w_ref[...], preferred_element_type=jnp.float32)
            * s,
            0.0,
        )

        @pl.when(j == N_DEV - 1)
        def _():
            out_ref[pl.ds(my * m_per, m_per), :] = blk

        @pl.when(j < N_DEV - 1)
        def _():
            sendbuf[j] = blk.astype(jnp.bfloat16)
            rdma = pltpu.make_async_remote_copy(
                src_ref=sendbuf.at[j],
                dst_ref=recvbuf.at[my],
                send_sem=send_sems.at[j],
                recv_sem=recv_sems.at[my],
                device_id=(tgt,),
                device_id_type=pl.DeviceIdType.MESH,
            )
            rdma.start()

        def wait_and_store(d):
            src = lax.rem(my + N_DEV - d, N_DEV)
            recv = pltpu.make_async_remote_copy(
                src_ref=sendbuf.at[0],
                dst_ref=recvbuf.at[src],
                send_sem=send_sems.at[0],
                recv_sem=recv_sems.at[src],
                device_id=(0,),
                device_id_type=pl.DeviceIdType.MESH,
            )
            recv.wait_recv()
            out_ref[pl.ds(src * m_per, m_per), :] = recvbuf[src].astype(
                jnp.float32
            )

        @pl.when(j == N_DEV - 2)
        def _():
            wait_and_store(1)

        @pl.when(j == N_DEV - 1)
        def _():
            wait_and_store(2)
            wait_and_store(3)
            for slot in range(N_DEV - 1):
                snd = pltpu.make_async_remote_copy(
                    src_ref=sendbuf.at[slot],
                    dst_ref=recvbuf.at[my],
                    send_sem=send_sems.at[slot],
                    recv_sem=recv_sems.at[my],
                    device_id=(0,),
                    device_id_type=pl.DeviceIdType.MESH,
                )
                snd.wait_send()

    grid_spec = pltpu.PrefetchScalarGridSpec(
        num_scalar_prefetch=1,
        grid=(N_DEV,),
        in_specs=[
            pl.BlockSpec((m_per, k), lambda j, my: (0, 0)),
            pl.BlockSpec(
                (k, n_per), lambda j, my: (0, lax.rem(my[0] + 1 + j, N_DEV))
            ),
            pl.BlockSpec(memory_space=pltpu.SMEM),
            pl.BlockSpec(memory_space=pltpu.SMEM),
        ],
        out_specs=pl.BlockSpec((N_DEV * m_per, n_per), lambda j, my: (0, 0)),
        scratch_shapes=[
            pltpu.VMEM((N_DEV - 1, m_per, n_per), jnp.bfloat16),
            pltpu.VMEM((N_DEV, m_per, n_per), jnp.bfloat16),
            pltpu.SemaphoreType.DMA((N_DEV - 1,)),
            pltpu.SemaphoreType.DMA((N_DEV,)),
        ],
    )

    out_shape = jax.ShapeDtypeStruct((N_DEV * m_per, n_per), jnp.float32)
    return pl.pallas_call(
        body,
        grid_spec=grid_spec,
        out_shape=out_shape,
        compiler_params=pltpu.CompilerParams(
            dimension_semantics=("arbitrary",),
            vmem_limit_bytes=63 * 1024 * 1024,
        ),
    )(my_arr, xq, wq_mat, scale_x, scale_w)


# device time: 53504 ns/iter; 1.3445x vs baseline; 1.3445x over previous
import jax
import jax.numpy as jnp
from jax import lax
from jax.experimental import pallas as pl
from jax.experimental.pallas import tpu as pltpu

N_DEV = 4
X_DT = jnp.float8_e4m3fn
W_DT = jnp.float8_e5m2
DIAG_J = 1


def _cast(a, dt):
    m, k = a.shape

    def body(a_ref, o_ref):
        o_ref[...] = a_ref[...].astype(dt)

    return pl.pallas_call(
        body,
        grid=(4,),
        in_specs=[pl.BlockSpec((m // 4, k), lambda i: (i, 0))],
        out_specs=pl.BlockSpec((m // 4, k), lambda i: (i, 0)),
        out_shape=jax.ShapeDtypeStruct((m, k), dt),
    )(a)


def kernel(x, w_mat, scale_x, scale_w):
    m_per, k = x.shape
    _, n = w_mat.shape
    n_per = n // N_DEV

    xq = _cast(x, X_DT)
    my_arr = jnp.full((1,), lax.axis_index("i"), jnp.int32)

    def body(my_ref, x_ref, w_ref, sx_ref, sw_ref, out_ref,
             sendbuf, recvbuf, diag_send, diag_recv, send_sems, recv_sems):
        j = pl.program_id(0)
        del my_ref
        my = lax.axis_index("i")
        tgt = lax.rem(my + 1 + j, N_DEV)
        s = sx_ref[0] * sw_ref[0]

        wq = w_ref[...].astype(W_DT)
        blk = jnp.maximum(
            jnp.dot(x_ref[...], wq, preferred_element_type=jnp.float32) * s,
            0.0,
        )

        @pl.when(j == N_DEV - 1)
        def _():
            out_ref[pl.ds(my * m_per, m_per), :] = blk

        @pl.when(jnp.logical_and(j < N_DEV - 1, j != DIAG_J))
        def _():
            sendbuf[j] = blk.astype(jnp.bfloat16)
            rdma = pltpu.make_async_remote_copy(
                src_ref=sendbuf.at[j],
                dst_ref=recvbuf.at[my],
                send_sem=send_sems.at[j],
                recv_sem=recv_sems.at[my],
                device_id=(tgt,),
                device_id_type=pl.DeviceIdType.MESH,
            )
            rdma.start()

        @pl.when(j == DIAG_J)
        def _():
            diag_send[...] = blk.astype(X_DT)
            rdma = pltpu.make_async_remote_copy(
                src_ref=diag_send,
                dst_ref=diag_recv,
                send_sem=send_sems.at[DIAG_J],
                recv_sem=recv_sems.at[my],
                device_id=(tgt,),
                device_id_type=pl.DeviceIdType.MESH,
            )
            rdma.start()

        def wait_and_store(d):
            src = lax.rem(my + N_DEV - d, N_DEV)
            dst = diag_recv if d == 2 else recvbuf.at[src]
            recv = pltpu.make_async_remote_copy(
                src_ref=diag_send if d == 2 else sendbuf.at[0],
                dst_ref=dst,
                send_sem=send_sems.at[0],
                recv_sem=recv_sems.at[src],
                device_id=(0,),
                device_id_type=pl.DeviceIdType.MESH,
            )
            recv.wait_recv()
            src_val = diag_recv[...] if d == 2 else recvbuf[src]
            out_ref[pl.ds(src * m_per, m_per), :] = src_val.astype(jnp.float32)

        @pl.when(j == N_DEV - 2)
        def _():
            wait_and_store(1)

        @pl.when(j == N_DEV - 1)
        def _():
            wait_and_store(2)
            wait_and_store(3)
            for slot in range(N_DEV - 1):
                snd = pltpu.make_async_remote_copy(
                    src_ref=diag_send if slot == DIAG_J else sendbuf.at[slot],
                    dst_ref=diag_recv if slot == DIAG_J else recvbuf.at[my],
                    send_sem=send_sems.at[slot],
                    recv_sem=recv_sems.at[my],
                    device_id=(0,),
                    device_id_type=pl.DeviceIdType.MESH,
                )
                snd.wait_send()

    grid_spec = pltpu.PrefetchScalarGridSpec(
        num_scalar_prefetch=1,
        grid=(N_DEV,),
        in_specs=[
            pl.BlockSpec((m_per, k), lambda j, my: (0, 0)),
            pl.BlockSpec(
                (k, n_per), lambda j, my: (0, lax.rem(my[0] + 1 + j, N_DEV))
            ),
            pl.BlockSpec(memory_space=pltpu.SMEM),
            pl.BlockSpec(memory_space=pltpu.SMEM),
        ],
        out_specs=pl.BlockSpec((N_DEV * m_per, n_per), lambda j, my: (0, 0)),
        scratch_shapes=[
            pltpu.VMEM((N_DEV - 1, m_per, n_per), jnp.bfloat16),
            pltpu.VMEM((N_DEV, m_per, n_per), jnp.bfloat16),
            pltpu.VMEM((m_per, n_per), X_DT),
            pltpu.VMEM((m_per, n_per), X_DT),
            pltpu.SemaphoreType.DMA((N_DEV - 1,)),
            pltpu.SemaphoreType.DMA((N_DEV,)),
        ],
    )

    out_shape = jax.ShapeDtypeStruct((N_DEV * m_per, n_per), jnp.float32)
    return pl.pallas_call(
        body,
        grid_spec=grid_spec,
        out_shape=out_shape,
        compiler_params=pltpu.CompilerParams(
            dimension_semantics=("arbitrary",),
            vmem_limit_bytes=63 * 1024 * 1024,
        ),
    )(my_arr, xq, w_mat, scale_x, scale_w)


# device time: 48883 ns/iter; 1.4716x vs baseline; 1.0945x over previous
import jax
import jax.numpy as jnp
from jax import lax
from jax.experimental import pallas as pl
from jax.experimental.pallas import tpu as pltpu

N_DEV = 4
X_DT = jnp.float8_e4m3fn
W_DT = jnp.float8_e5m2


def _cast(a, dt):
    m, k = a.shape

    def body(a_ref, o_ref):
        o_ref[...] = a_ref[...].astype(dt)

    return pl.pallas_call(
        body,
        grid=(4,),
        in_specs=[pl.BlockSpec((m // 4, k), lambda i: (i, 0))],
        out_specs=pl.BlockSpec((m // 4, k), lambda i: (i, 0)),
        out_shape=jax.ShapeDtypeStruct((m, k), dt),
    )(a)


def kernel(x, w_mat, scale_x, scale_w):
    m_per, k = x.shape
    _, n = w_mat.shape
    n_per = n // N_DEV

    xq = _cast(x, X_DT)
    my_arr = jnp.full((1,), lax.axis_index("i"), jnp.int32)

    def body(my_ref, x_ref, w_ref, sx_ref, sw_ref, out_ref,
             sendbuf, recvbuf, ssend, srecv,
             send_sems, recv_sems, ssend_sems, srecv_sems):
        j = pl.program_id(0)
        del my_ref
        my = lax.axis_index("i")
        tgt = lax.rem(my + 1 + j, N_DEV)
        s = sx_ref[0] * sw_ref[0]

        wq = w_ref[...].astype(W_DT)
        blk = jnp.maximum(
            jnp.dot(x_ref[...], wq, preferred_element_type=jnp.float32) * s,
            0.0,
        )

        @pl.when(j == N_DEV - 1)
        def _():
            out_ref[pl.ds(my * m_per, m_per), :] = blk

        @pl.when(j < N_DEV - 1)
        def _():
            cmax = jnp.max(blk, axis=0, keepdims=True)
            ssend[j] = jnp.maximum(cmax, 1e-30) * (1.0 / 127.0)
            sendbuf[j] = jnp.rint(blk * (127.0 / jnp.maximum(cmax, 1e-30))
                                  ).astype(jnp.int8)
            rdma = pltpu.make_async_remote_copy(
                src_ref=sendbuf.at[j],
                dst_ref=recvbuf.at[my],
                send_sem=send_sems.at[j],
                recv_sem=recv_sems.at[my],
                device_id=(tgt,),
                device_id_type=pl.DeviceIdType.MESH,
            )
            rdma.start()
            srdma = pltpu.make_async_remote_copy(
                src_ref=ssend.at[j],
                dst_ref=srecv.at[my],
                send_sem=ssend_sems.at[j],
                recv_sem=srecv_sems.at[my],
                device_id=(tgt,),
                device_id_type=pl.DeviceIdType.MESH,
            )
            srdma.start()

        def wait_and_store(d):
            src = lax.rem(my + N_DEV - d, N_DEV)
            recv = pltpu.make_async_remote_copy(
                src_ref=sendbuf.at[0],
                dst_ref=recvbuf.at[src],
                send_sem=send_sems.at[0],
                recv_sem=recv_sems.at[src],
                device_id=(0,),
                device_id_type=pl.DeviceIdType.MESH,
            )
            recv.wait_recv()
            srecv_d = pltpu.make_async_remote_copy(
                src_ref=ssend.at[0],
                dst_ref=srecv.at[src],
                send_sem=ssend_sems.at[0],
                recv_sem=srecv_sems.at[src],
                device_id=(0,),
                device_id_type=pl.DeviceIdType.MESH,
            )
            srecv_d.wait_recv()
            out_ref[pl.ds(src * m_per, m_per), :] = (
                recvbuf[src].astype(jnp.float32) * srecv[src]
            )

        @pl.when(j == N_DEV - 2)
        def _():
            wait_and_store(1)

        @pl.when(j == N_DEV - 1)
        def _():
            wait_and_store(2)
            wait_and_store(3)
            for slot in range(N_DEV - 1):
                snd = pltpu.make_async_remote_copy(
                    src_ref=sendbuf.at[slot],
                    dst_ref=recvbuf.at[my],
                    send_sem=send_sems.at[slot],
                    recv_sem=recv_sems.at[my],
                    device_id=(0,),
                    device_id_type=pl.DeviceIdType.MESH,
                )
                snd.wait_send()
                ssnd = pltpu.make_async_remote_copy(
                    src_ref=ssend.at[slot],
                    dst_ref=srecv.at[my],
                    send_sem=ssend_sems.at[slot],
                    recv_sem=srecv_sems.at[my],
                    device_id=(0,),
                    device_id_type=pl.DeviceIdType.MESH,
                )
                ssnd.wait_send()

    grid_spec = pltpu.PrefetchScalarGridSpec(
        num_scalar_prefetch=1,
        grid=(N_DEV,),
        in_specs=[
            pl.BlockSpec((m_per, k), lambda j, my: (0, 0)),
            pl.BlockSpec(
                (k, n_per), lambda j, my: (0, lax.rem(my[0] + 1 + j, N_DEV))
            ),
            pl.BlockSpec(memory_space=pltpu.SMEM),
            pl.BlockSpec(memory_space=pltpu.SMEM),
        ],
        out_specs=pl.BlockSpec((N_DEV * m_per, n_per), lambda j, my: (0, 0)),
        scratch_shapes=[
            pltpu.VMEM((N_DEV - 1, m_per, n_per), jnp.int8),
            pltpu.VMEM((N_DEV, m_per, n_per), jnp.int8),
            pltpu.VMEM((N_DEV - 1, 1, n_per), jnp.float32),
            pltpu.VMEM((N_DEV, 1, n_per), jnp.float32),
            pltpu.SemaphoreType.DMA((N_DEV - 1,)),
            pltpu.SemaphoreType.DMA((N_DEV,)),
            pltpu.SemaphoreType.DMA((N_DEV - 1,)),
            pltpu.SemaphoreType.DMA((N_DEV,)),
        ],
    )

    out_shape = jax.ShapeDtypeStruct((N_DEV * m_per, n_per), jnp.float32)
    return pl.pallas_call(
        body,
        grid_spec=grid_spec,
        out_shape=out_shape,
        compiler_params=pltpu.CompilerParams(
            dimension_semantics=("arbitrary",),
            vmem_limit_bytes=63 * 1024 * 1024,
        ),
    )(my_arr, xq, w_mat, scale_x, scale_w)


# device time: 46517 ns/iter; 1.5465x vs baseline; 1.0509x over previous
import jax
import jax.numpy as jnp
from jax import lax
from jax.experimental import pallas as pl
from jax.experimental.pallas import tpu as pltpu

N_DEV = 4
X_DT = jnp.float8_e4m3fn
W_DT = jnp.float8_e5m2


def _cast(a, dt):
    m, k = a.shape

    def body(a_ref, o_ref):
        o_ref[...] = a_ref[...].astype(dt)

    return pl.pallas_call(
        body,
        grid=(4,),
        in_specs=[pl.BlockSpec((m // 4, k), lambda i: (i, 0))],
        out_specs=pl.BlockSpec((m // 4, k), lambda i: (i, 0)),
        out_shape=jax.ShapeDtypeStruct((m, k), dt),
    )(a)


def kernel(x, w_mat, scale_x, scale_w):
    m_per, k = x.shape
    _, n = w_mat.shape
    n_per = n // N_DEV

    xq = _cast(x, X_DT)
    my_arr = jnp.full((1,), lax.axis_index("i"), jnp.int32)

    def body(my_ref, x_ref, w_ref, sx_ref, sw_ref, out_ref,
             sendbuf, recvbuf, ssend, srecv, stage,
             send_sems, recv_sems, ssend_sems, srecv_sems, copy_sems):
        j = pl.program_id(0)
        del my_ref
        my = lax.axis_index("i")
        tgt = lax.rem(my + 1 + j, N_DEV)
        s = sx_ref[0] * sw_ref[0]

        wq = w_ref[...].astype(W_DT)
        blk = jnp.maximum(
            jnp.dot(x_ref[...], wq, preferred_element_type=jnp.float32) * s,
            0.0,
        )

        @pl.when(j == N_DEV - 1)
        def _():
            stage[N_DEV - 1] = blk
            pltpu.make_async_copy(
                stage.at[N_DEV - 1],
                out_ref.at[pl.ds(my * m_per, m_per), :],
                copy_sems.at[N_DEV - 1],
            ).start()

        @pl.when(j < N_DEV - 1)
        def _():
            cmax = jnp.max(blk, axis=0, keepdims=True)
            ssend[j] = jnp.maximum(cmax, 1e-30) * (1.0 / 127.0)
            sendbuf[j] = jnp.rint(blk * (127.0 / jnp.maximum(cmax, 1e-30))
                                  ).astype(jnp.int8)
            rdma = pltpu.make_async_remote_copy(
                src_ref=sendbuf.at[j],
                dst_ref=recvbuf.at[my],
                send_sem=send_sems.at[j],
                recv_sem=recv_sems.at[my],
                device_id=(tgt,),
                device_id_type=pl.DeviceIdType.MESH,
            )
            rdma.start()
            srdma = pltpu.make_async_remote_copy(
                src_ref=ssend.at[j],
                dst_ref=srecv.at[my],
                send_sem=ssend_sems.at[j],
                recv_sem=srecv_sems.at[my],
                device_id=(tgt,),
                device_id_type=pl.DeviceIdType.MESH,
            )
            srdma.start()

        def wait_and_store(d):
            src = lax.rem(my + N_DEV - d, N_DEV)
            recv = pltpu.make_async_remote_copy(
                src_ref=sendbuf.at[0],
                dst_ref=recvbuf.at[src],
                send_sem=send_sems.at[0],
                recv_sem=recv_sems.at[src],
                device_id=(0,),
                device_id_type=pl.DeviceIdType.MESH,
            )
            recv.wait_recv()
            srecv_d = pltpu.make_async_remote_copy(
                src_ref=ssend.at[0],
                dst_ref=srecv.at[src],
                send_sem=ssend_sems.at[0],
                recv_sem=srecv_sems.at[src],
                device_id=(0,),
                device_id_type=pl.DeviceIdType.MESH,
            )
            srecv_d.wait_recv()
            stage[d - 1] = recvbuf[src].astype(jnp.float32) * srecv[src]
            pltpu.make_async_copy(
                stage.at[d - 1],
                out_ref.at[pl.ds(src * m_per, m_per), :],
                copy_sems.at[d - 1],
            ).start()

        @pl.when(j == N_DEV - 2)
        def _():
            wait_and_store(1)

        @pl.when(j == N_DEV - 1)
        def _():
            wait_and_store(2)
            wait_and_store(3)
            for slot in range(N_DEV - 1):
                snd = pltpu.make_async_remote_copy(
                    src_ref=sendbuf.at[slot],
                    dst_ref=recvbuf.at[my],
                    send_sem=send_sems.at[slot],
                    recv_sem=recv_sems.at[my],
                    device_id=(0,),
                    device_id_type=pl.DeviceIdType.MESH,
                )
                snd.wait_send()
                ssnd = pltpu.make_async_remote_copy(
                    src_ref=ssend.at[slot],
                    dst_ref=srecv.at[my],
                    send_sem=ssend_sems.at[slot],
                    recv_sem=srecv_sems.at[my],
                    device_id=(0,),
                    device_id_type=pl.DeviceIdType.MESH,
                )
                ssnd.wait_send()

            for slot in range(N_DEV):
                rows = (
                    my if slot == N_DEV - 1
                    else lax.rem(my + N_DEV - 1 - slot, N_DEV)
                )
                pltpu.make_async_copy(
                    stage.at[slot],
                    out_ref.at[pl.ds(rows * m_per, m_per), :],
                    copy_sems.at[slot],
                ).wait()

    grid_spec = pltpu.PrefetchScalarGridSpec(
        num_scalar_prefetch=1,
        grid=(N_DEV,),
        in_specs=[
            pl.BlockSpec((m_per, k), lambda j, my: (0, 0)),
            pl.BlockSpec(
                (k, n_per), lambda j, my: (0, lax.rem(my[0] + 1 + j, N_DEV))
            ),
            pl.BlockSpec(memory_space=pltpu.SMEM),
            pl.BlockSpec(memory_space=pltpu.SMEM),
        ],
        out_specs=pl.BlockSpec(memory_space=pl.ANY),
        scratch_shapes=[
            pltpu.VMEM((N_DEV - 1, m_per, n_per), jnp.int8),
            pltpu.VMEM((N_DEV, m_per, n_per), jnp.int8),
            pltpu.VMEM((N_DEV - 1, 1, n_per), jnp.float32),
            pltpu.VMEM((N_DEV, 1, n_per), jnp.float32),
            pltpu.VMEM((N_DEV, m_per, n_per), jnp.float32),
            pltpu.SemaphoreType.DMA((N_DEV - 1,)),
            pltpu.SemaphoreType.DMA((N_DEV,)),
            pltpu.SemaphoreType.DMA((N_DEV - 1,)),
            pltpu.SemaphoreType.DMA((N_DEV,)),
            pltpu.SemaphoreType.DMA((N_DEV,)),
        ],
    )

    out_shape = jax.ShapeDtypeStruct((N_DEV * m_per, n_per), jnp.float32)
    return pl.pallas_call(
        body,
        grid_spec=grid_spec,
        out_shape=out_shape,
        compiler_params=pltpu.CompilerParams(
            dimension_semantics=("arbitrary",),
            vmem_limit_bytes=63 * 1024 * 1024,
        ),
    )(my_arr, xq, w_mat, scale_x, scale_w)
